# baseline (speedup 1.0000x reference)
import functools

import jax
import jax.numpy as jnp
from jax import lax
from jax.experimental import pallas as pl
from jax.experimental.pallas import tpu as pltpu
from jax.experimental.pallas import tpu_sc as plsc

BATCH = 4096
MAXLEN = 200
EMBED = 64
LANES = 16
EGROUPS = EMBED // 8
VGROUPS = EMBED // LANES

NUM_CORES = 2
NUM_SUBCORES = 16
NUM_WORKERS = NUM_CORES * NUM_SUBCORES
BTILES = BATCH // 128

SPAN = 2
NTASKS = MAXLEN // SPAN
NROWS = SPAN * 128
GDEPTH = 3
WDEPTH = 2


def _body(xt_hbm, tok_hbm, pos_hbm, out_hbm,
          idx0, idx1, idx2, rows0, rows1, rows2, t0, t1, pos_v,
          gsem, isem, osem):
    idx_v = (idx0, idx1, idx2)
    rows_v = (rows0, rows1, rows2)
    tout = (t0, t1)
    wid = lax.axis_index("s") * NUM_CORES + lax.axis_index("c")

    pltpu.sync_copy(pos_hbm, pos_v)

    def idx_start(t, p):
        return pltpu.async_copy(
            xt_hbm.at[pl.ds((t * SPAN) * BATCH + wid * 128, 128)],
            idx_v[p].at[pl.ds(0, 128)], isem.at[p])

    def idx_start2(t, p):
        return pltpu.async_copy(
            xt_hbm.at[pl.ds((t * SPAN + 1) * BATCH + wid * 128, 128)],
            idx_v[p].at[pl.ds(128, 128)], isem.at[p])

    def idx_wait_all(p):
        pltpu.make_async_copy(
            xt_hbm.at[pl.ds(0, NROWS)], idx_v[p], isem.at[p]).wait()

    def gather_start(p):
        return pltpu.async_copy(tok_hbm.at[idx_v[p]], rows_v[p], gsem.at[p])

    def gather_wait(p):
        pltpu.make_async_copy(tok_hbm.at[idx_v[p]], rows_v[p], gsem.at[p]).wait()

    def wb_start(t, p):
        c = []
        for j in range(SPAN):
            c.append(pltpu.async_copy(
                tout[p].at[j], out_hbm.at[t * SPAN + j, :, wid], osem.at[p]))
        return c

    def wb_wait(p):
        pltpu.make_async_copy(
            out_hbm.at[pl.ds(0, SPAN), :, wid], tout[p], osem.at[p]).wait()

    iota = lax.iota(jnp.int32, LANES)
    et_ids = []
    inner_base = []
    for g in range(VGROUPS):
        e = iota + (g * LANES)
        et_ids.append(e >> 3)
        inner_base.append((e & 7) << 7)

    def transpose_add(t, p):
        for j in range(SPAN):
            s = t * SPAN + j
            pos_regs = [pos_v[s, pl.ds(g * LANES, LANES)] for g in range(VGROUPS)]

            @plsc.parallel_loop(0, 128, unroll=4)
            def _(b):
                b_splat = jnp.full((LANES,), b, dtype=jnp.int32)
                for g in range(VGROUPS):
                    v = rows_v[p][j * 128 + b, pl.ds(g * LANES, LANES)] + pos_regs[g]
                    plsc.store_scatter(
                        tout[p].at[j], [et_ids[g], inner_base[g] + b_splat], v)

    for t in range(GDEPTH):
        idx_start(t, t)
        idx_start2(t, t)
    for t in range(GDEPTH):
        idx_wait_all(t)
        gather_start(t)

    def task_body(t, p, pw, prefetch, drain):
        gather_wait(p)
        if prefetch:
            idx_start(t + GDEPTH, p)
            idx_start2(t + GDEPTH, p)
        if drain:
            wb_wait(pw)
        transpose_add(t, pw)
        wb_start(t, pw)
        if prefetch:
            idx_wait_all(p)
            gather_start(p)

    for t in range(WDEPTH):
        task_body(t, t % GDEPTH, t % WDEPTH, True, False)

    STEADY0 = WDEPTH
    STEADY1 = NTASKS - GDEPTH
    NGRP = (STEADY1 - STEADY0) // 6

    def grp_body(gg, _):
        base = STEADY0 + gg * 6
        for k in range(6):
            t = base + k
            task_body(t, (STEADY0 + k) % GDEPTH, k % WDEPTH, True, True)
        return 0

    lax.fori_loop(0, NGRP, grp_body, 0)

    for t in range(STEADY0 + NGRP * 6, STEADY1):
        task_body(t, t % GDEPTH, t % WDEPTH, True, True)

    for t in range(STEADY1, NTASKS):
        task_body(t, t % GDEPTH, t % WDEPTH, False, True)

    for t in range(NTASKS - WDEPTH, NTASKS):
        wb_wait(t % WDEPTH)


@jax.jit
def _embed(xt_flat, token_table, pos_table):
    mesh = plsc.VectorSubcoreMesh(core_axis_name="c", subcore_axis_name="s")
    k = functools.partial(
        pl.kernel,
        mesh=mesh,
        out_type=jax.ShapeDtypeStruct((MAXLEN, EGROUPS, BTILES, 1024), jnp.float32),
        scratch_types=[
            pltpu.VMEM((NROWS,), jnp.int32),
            pltpu.VMEM((NROWS,), jnp.int32),
            pltpu.VMEM((NROWS,), jnp.int32),
            pltpu.VMEM((NROWS, EMBED), jnp.float32),
            pltpu.VMEM((NROWS, EMBED), jnp.float32),
            pltpu.VMEM((NROWS, EMBED), jnp.float32),
            pltpu.VMEM((SPAN, EGROUPS, 1024), jnp.float32),
            pltpu.VMEM((SPAN, EGROUPS, 1024), jnp.float32),
            pltpu.VMEM((MAXLEN, EMBED), jnp.float32),
            pltpu.SemaphoreType.DMA((GDEPTH,)),
            pltpu.SemaphoreType.DMA((GDEPTH,)),
            pltpu.SemaphoreType.DMA((WDEPTH,)),
        ],
        compiler_params=pltpu.CompilerParams(use_tc_tiling_on_sc=False, needs_layout_passes=False),
    )(_body)
    return k(xt_flat, token_table, pos_table)


def kernel(x, token_table, pos_table):
    xt_flat = jnp.transpose(x).reshape(-1).astype(jnp.int32)
    out5 = _embed(xt_flat, token_table, pos_table)
    out = (out5.reshape(MAXLEN, EGROUPS, BTILES, 8, 128)
           .transpose(2, 4, 0, 1, 3).reshape(BATCH, MAXLEN, EMBED))
    return out


# speedup vs baseline: 1.0001x; 1.0001x over previous
import functools

import jax
import jax.numpy as jnp
from jax import lax
from jax.experimental import pallas as pl
from jax.experimental.pallas import tpu as pltpu
from jax.experimental.pallas import tpu_sc as plsc

BATCH = 4096
MAXLEN = 200
EMBED = 64
LANES = 16
EGROUPS = EMBED // 8
VGROUPS = EMBED // LANES

NUM_CORES = 2
NUM_SUBCORES = 16
NUM_WORKERS = NUM_CORES * NUM_SUBCORES
BTILES = BATCH // 128

SPAN = 2
NTASKS = MAXLEN // SPAN
NROWS = SPAN * 128
GDEPTH = 3
WDEPTH = 2


def _body(xt_hbm, tok_hbm, pos_hbm, out_hbm,
          idx0, idx1, idx2, rows0, rows1, rows2, t0, t1, pos_v,
          gsem, isem, osem):
    idx_v = (idx0, idx1, idx2)
    rows_v = (rows0, rows1, rows2)
    tout = (t0, t1)
    wid = lax.axis_index("s") * NUM_CORES + lax.axis_index("c")

    pltpu.sync_copy(pos_hbm, pos_v)

    def idx_start(t, p):
        for j in range(SPAN):
            pltpu.async_copy(
                xt_hbm.at[pl.ds((t * SPAN + j) * BATCH + wid * 128, 128)],
                idx_v[p].at[j], isem.at[p])

    def idx_wait_all(p):
        for j in range(SPAN):
            pltpu.make_async_copy(
                xt_hbm.at[pl.ds(j * BATCH, 128)], idx_v[p].at[j],
                isem.at[p]).wait()

    def gather_start(p):
        for j in range(SPAN):
            pltpu.async_copy(
                tok_hbm.at[idx_v[p].at[j]],
                rows_v[p].at[pl.ds(j * 128, 128)], gsem.at[p])

    def gather_wait(p):
        for j in range(SPAN):
            pltpu.make_async_copy(
                tok_hbm.at[idx_v[p].at[j]],
                rows_v[p].at[pl.ds(j * 128, 128)], gsem.at[p]).wait()

    def wb_start(t, p):
        c = []
        for j in range(SPAN):
            c.append(pltpu.async_copy(
                tout[p].at[j], out_hbm.at[t * SPAN + j, :, wid], osem.at[p]))
        return c

    def wb_wait(p):
        pltpu.make_async_copy(
            out_hbm.at[pl.ds(0, SPAN), :, wid], tout[p], osem.at[p]).wait()

    iota = lax.iota(jnp.int32, LANES)
    et_ids = []
    inner_base = []
    for g in range(VGROUPS):
        e = iota + (g * LANES)
        et_ids.append(e >> 3)
        inner_base.append((e & 7) << 7)

    def transpose_add(t, pg, pw):
        for j in range(SPAN):
            s = t * SPAN + j
            pos_regs = [pos_v[s, pl.ds(g * LANES, LANES)] for g in range(VGROUPS)]

            @plsc.parallel_loop(0, 128, unroll=4)
            def _(b):
                b_splat = jnp.full((LANES,), b, dtype=jnp.int32)
                for g in range(VGROUPS):
                    v = rows_v[pg][j * 128 + b, pl.ds(g * LANES, LANES)] + pos_regs[g]
                    plsc.store_scatter(
                        tout[pw].at[j], [et_ids[g], inner_base[g] + b_splat], v)

    for t in range(GDEPTH):
        idx_start(t, t)
    for t in range(GDEPTH):
        idx_wait_all(t)
        gather_start(t)

    def task_body(t, p, pw, prefetch, drain):
        gather_wait(p)
        if prefetch:
            idx_start(t + GDEPTH, p)
        if drain:
            wb_wait(pw)
        transpose_add(t, p, pw)
        wb_start(t, pw)
        if prefetch:
            idx_wait_all(p)
            gather_start(p)

    for t in range(WDEPTH):
        task_body(t, t % GDEPTH, t % WDEPTH, True, False)

    STEADY0 = WDEPTH
    STEADY1 = NTASKS - GDEPTH
    NGRP = (STEADY1 - STEADY0) // 6

    def grp_body(gg, _):
        base = STEADY0 + gg * 6
        for k in range(6):
            t = base + k
            task_body(t, (STEADY0 + k) % GDEPTH, k % WDEPTH, True, True)
        return 0

    lax.fori_loop(0, NGRP, grp_body, 0)

    for t in range(STEADY0 + NGRP * 6, STEADY1):
        task_body(t, t % GDEPTH, t % WDEPTH, True, True)

    for t in range(STEADY1, NTASKS):
        task_body(t, t % GDEPTH, t % WDEPTH, False, True)

    for t in range(NTASKS - WDEPTH, NTASKS):
        wb_wait(t % WDEPTH)


@jax.jit
def _embed(xt_flat, token_table, pos_table):
    mesh = plsc.VectorSubcoreMesh(core_axis_name="c", subcore_axis_name="s")
    k = functools.partial(
        pl.kernel,
        mesh=mesh,
        out_type=jax.ShapeDtypeStruct((MAXLEN, EGROUPS, BTILES, 1024), jnp.float32),
        scratch_types=[
            pltpu.VMEM((SPAN, 128), jnp.int32),
            pltpu.VMEM((SPAN, 128), jnp.int32),
            pltpu.VMEM((SPAN, 128), jnp.int32),
            pltpu.VMEM((NROWS, EMBED), jnp.float32),
            pltpu.VMEM((NROWS, EMBED), jnp.float32),
            pltpu.VMEM((NROWS, EMBED), jnp.float32),
            pltpu.VMEM((SPAN, EGROUPS, 1024), jnp.float32),
            pltpu.VMEM((SPAN, EGROUPS, 1024), jnp.float32),
            pltpu.VMEM((MAXLEN, EMBED), jnp.float32),
            pltpu.SemaphoreType.DMA((GDEPTH,)),
            pltpu.SemaphoreType.DMA((GDEPTH,)),
            pltpu.SemaphoreType.DMA((WDEPTH,)),
        ],
        compiler_params=pltpu.CompilerParams(use_tc_tiling_on_sc=False, needs_layout_passes=False),
    )(_body)
    return k(xt_flat, token_table, pos_table)


def kernel(x, token_table, pos_table):
    xt_flat = jnp.transpose(x).reshape(-1).astype(jnp.int32)
    out5 = _embed(xt_flat, token_table, pos_table)
    out = (out5.reshape(MAXLEN, EGROUPS, BTILES, 8, 128)
           .transpose(2, 4, 0, 1, 3).reshape(BATCH, MAXLEN, EMBED))
    return out


# speedup vs baseline: 1.0603x; 1.0601x over previous
import functools

import jax
import jax.numpy as jnp
from jax import lax
from jax.experimental import pallas as pl
from jax.experimental.pallas import tpu as pltpu
from jax.experimental.pallas import tpu_sc as plsc

BATCH = 4096
MAXLEN = 200
EMBED = 64
LANES = 16
VGROUPS = EMBED // LANES

NUM_CORES = 2
NUM_SUBCORES = 16

SPAN = 2
NTASKS = MAXLEN // SPAN
NROWS = SPAN * 128
DEPTH = 5
PRE = 3


def _body(xt_hbm, tok_hbm, pos_hbm, out_hbm,
          idx0, idx1, idx2, idx3, idx4,
          buf0, buf1, buf2, buf3, buf4, pos_v,
          gsem, isem, osem):
    idx_v = (idx0, idx1, idx2, idx3, idx4)
    bufs = (buf0, buf1, buf2, buf3, buf4)
    wid = lax.axis_index("s") * NUM_CORES + lax.axis_index("c")

    pltpu.sync_copy(pos_hbm, pos_v)

    def idx_start(t, p):
        for j in range(SPAN):
            pltpu.async_copy(
                xt_hbm.at[pl.ds((t * SPAN + j) * BATCH + wid * 128, 128)],
                idx_v[p].at[j], isem.at[p])

    def idx_wait_all(p):
        for j in range(SPAN):
            pltpu.make_async_copy(
                xt_hbm.at[pl.ds(j * BATCH, 128)], idx_v[p].at[j],
                isem.at[p]).wait()

    def gather_start(p):
        for j in range(SPAN):
            pltpu.async_copy(
                tok_hbm.at[idx_v[p].at[j]],
                bufs[p].at[pl.ds(j * 128, 128)], gsem.at[p])

    def gather_wait(p):
        for j in range(SPAN):
            pltpu.make_async_copy(
                tok_hbm.at[idx_v[p].at[j]],
                bufs[p].at[pl.ds(j * 128, 128)], gsem.at[p]).wait()

    def wb_start(t, p):
        for j in range(SPAN):
            pltpu.async_copy(
                bufs[p].at[pl.ds(j * 128, 128)],
                out_hbm.at[pl.ds(wid * 128, 128), t * SPAN + j], osem.at[p])

    def wb_wait(p):
        for j in range(SPAN):
            pltpu.make_async_copy(
                bufs[p].at[pl.ds(j * 128, 128)],
                out_hbm.at[pl.ds(wid * 128, 128), j], osem.at[p]).wait()

    def add_pos(t, p):
        for j in range(SPAN):
            s = t * SPAN + j
            pos_regs = [pos_v[s, pl.ds(g * LANES, LANES)] for g in range(VGROUPS)]

            @plsc.parallel_loop(0, 128, unroll=4)
            def _(b):
                for g in range(VGROUPS):
                    plsc.addupdate(
                        bufs[p].at[j * 128 + b, pl.ds(g * LANES, LANES)],
                        pos_regs[g])

    for t in range(PRE):
        idx_start(t, t)
    for t in range(PRE):
        idx_wait_all(t)
        gather_start(t)

    def task_body(t, p, p2, prefetch, drain):
        gather_wait(p)
        if prefetch:
            idx_start(t + PRE, p2)
        add_pos(t, p)
        wb_start(t, p)
        if prefetch:
            if drain:
                wb_wait(p2)
            idx_wait_all(p2)
            gather_start(p2)

    for t in range(2):
        task_body(t, t % DEPTH, (t + PRE) % DEPTH, True, False)

    def grp_body(gg, _):
        base = 2 + gg * DEPTH
        for k in range(DEPTH):
            t = base + k
            task_body(t, (2 + k) % DEPTH, k % DEPTH, True, True)
        return 0

    lax.fori_loop(0, 19, grp_body, 0)

    for t in range(97, NTASKS):
        task_body(t, t % DEPTH, (t + PRE) % DEPTH, False, False)

    for t in range(NTASKS - DEPTH, NTASKS):
        wb_wait(t % DEPTH)


@jax.jit
def _embed(xt_flat, token_table, pos_table):
    mesh = plsc.VectorSubcoreMesh(core_axis_name="c", subcore_axis_name="s")
    k = functools.partial(
        pl.kernel,
        mesh=mesh,
        out_type=jax.ShapeDtypeStruct((BATCH, MAXLEN, EMBED), jnp.float32),
        scratch_types=[
            pltpu.VMEM((SPAN, 128), jnp.int32),
            pltpu.VMEM((SPAN, 128), jnp.int32),
            pltpu.VMEM((SPAN, 128), jnp.int32),
            pltpu.VMEM((SPAN, 128), jnp.int32),
            pltpu.VMEM((SPAN, 128), jnp.int32),
            pltpu.VMEM((NROWS, EMBED), jnp.float32),
            pltpu.VMEM((NROWS, EMBED), jnp.float32),
            pltpu.VMEM((NROWS, EMBED), jnp.float32),
            pltpu.VMEM((NROWS, EMBED), jnp.float32),
            pltpu.VMEM((NROWS, EMBED), jnp.float32),
            pltpu.VMEM((MAXLEN, EMBED), jnp.float32),
            pltpu.SemaphoreType.DMA((DEPTH,)),
            pltpu.SemaphoreType.DMA((DEPTH,)),
            pltpu.SemaphoreType.DMA((DEPTH,)),
        ],
        compiler_params=pltpu.CompilerParams(use_tc_tiling_on_sc=False, needs_layout_passes=False),
    )(_body)
    return k(xt_flat, token_table, pos_table)


def kernel(x, token_table, pos_table):
    xt_flat = jnp.transpose(x).reshape(-1).astype(jnp.int32)
    return _embed(xt_flat, token_table, pos_table)


# speedup vs baseline: 1.0624x; 1.0020x over previous
import functools

import jax
import jax.numpy as jnp
from jax import lax
from jax.experimental import pallas as pl
from jax.experimental.pallas import tpu as pltpu
from jax.experimental.pallas import tpu_sc as plsc

BATCH = 4096
MAXLEN = 200
EMBED = 64
LANES = 16
VGROUPS = EMBED // LANES
BCHUNKS = 128 // LANES

NUM_CORES = 2
NUM_SUBCORES = 16

SPAN = 2
NTASKS = MAXLEN // SPAN
NROWS = SPAN * 128
DEPTH = 5
PRE = 3


def _body(x_hbm, tok_hbm, pos_hbm, out_hbm,
          idx0, idx1, idx2, idx3, idx4,
          buf0, buf1, buf2, buf3, buf4, x_v, pos_v,
          gsem, osem):
    idx_v = (idx0, idx1, idx2, idx3, idx4)
    bufs = (buf0, buf1, buf2, buf3, buf4)
    wid = lax.axis_index("s") * NUM_CORES + lax.axis_index("c")

    pltpu.sync_copy(x_hbm.at[pl.ds(wid * 128, 128)], x_v)
    pltpu.sync_copy(pos_hbm, pos_v)

    iota = lax.iota(jnp.int32, LANES)
    b_ids = [iota + (c * LANES) for c in range(BCHUNKS)]

    def idx_fill(t, p):
        for j in range(SPAN):
            s_splat = jnp.full((LANES,), t * SPAN + j, dtype=jnp.int32)
            for c in range(BCHUNKS):
                v = plsc.load_gather(x_v, [b_ids[c], s_splat])
                idx_v[p][j, pl.ds(c * LANES, LANES)] = v

    def gather_start(p):
        for j in range(SPAN):
            pltpu.async_copy(
                tok_hbm.at[idx_v[p].at[j]],
                bufs[p].at[pl.ds(j * 128, 128)], gsem.at[p])

    def gather_wait(p):
        for j in range(SPAN):
            pltpu.make_async_copy(
                tok_hbm.at[idx_v[p].at[j]],
                bufs[p].at[pl.ds(j * 128, 128)], gsem.at[p]).wait()

    def wb_start(t, p):
        for j in range(SPAN):
            pltpu.async_copy(
                bufs[p].at[pl.ds(j * 128, 128)],
                out_hbm.at[pl.ds(wid * 128, 128), t * SPAN + j], osem.at[p])

    def wb_wait(p):
        for j in range(SPAN):
            pltpu.make_async_copy(
                bufs[p].at[pl.ds(j * 128, 128)],
                out_hbm.at[pl.ds(wid * 128, 128), j], osem.at[p]).wait()

    def add_pos(t, p):
        for j in range(SPAN):
            s = t * SPAN + j
            pos_regs = [pos_v[s, pl.ds(g * LANES, LANES)] for g in range(VGROUPS)]

            @plsc.parallel_loop(0, 128, unroll=4)
            def _(b):
                for g in range(VGROUPS):
                    plsc.addupdate(
                        bufs[p].at[j * 128 + b, pl.ds(g * LANES, LANES)],
                        pos_regs[g])

    for t in range(PRE):
        idx_fill(t, t)
        gather_start(t)

    def task_body(t, p, p2, prefetch, drain):
        gather_wait(p)
        if prefetch:
            idx_fill(t + PRE, p2)
        add_pos(t, p)
        wb_start(t, p)
        if prefetch:
            if drain:
                wb_wait(p2)
            gather_start(p2)

    for t in range(2):
        task_body(t, t % DEPTH, (t + PRE) % DEPTH, True, False)

    def grp_body(gg, _):
        base = 2 + gg * DEPTH
        for k in range(DEPTH):
            t = base + k
            task_body(t, (2 + k) % DEPTH, k % DEPTH, True, True)
        return 0

    lax.fori_loop(0, 19, grp_body, 0)

    for t in range(97, NTASKS):
        task_body(t, t % DEPTH, (t + PRE) % DEPTH, False, False)

    for t in range(NTASKS - DEPTH, NTASKS):
        wb_wait(t % DEPTH)


@jax.jit
def _embed(x, token_table, pos_table):
    mesh = plsc.VectorSubcoreMesh(core_axis_name="c", subcore_axis_name="s")
    k = functools.partial(
        pl.kernel,
        mesh=mesh,
        out_type=jax.ShapeDtypeStruct((BATCH, MAXLEN, EMBED), jnp.float32),
        scratch_types=[
            pltpu.VMEM((SPAN, 128), jnp.int32),
            pltpu.VMEM((SPAN, 128), jnp.int32),
            pltpu.VMEM((SPAN, 128), jnp.int32),
            pltpu.VMEM((SPAN, 128), jnp.int32),
            pltpu.VMEM((SPAN, 128), jnp.int32),
            pltpu.VMEM((NROWS, EMBED), jnp.float32),
            pltpu.VMEM((NROWS, EMBED), jnp.float32),
            pltpu.VMEM((NROWS, EMBED), jnp.float32),
            pltpu.VMEM((NROWS, EMBED), jnp.float32),
            pltpu.VMEM((NROWS, EMBED), jnp.float32),
            pltpu.VMEM((128, MAXLEN), jnp.int32),
            pltpu.VMEM((MAXLEN, EMBED), jnp.float32),
            pltpu.SemaphoreType.DMA((DEPTH,)),
            pltpu.SemaphoreType.DMA((DEPTH,)),
        ],
        compiler_params=pltpu.CompilerParams(use_tc_tiling_on_sc=False, needs_layout_passes=False),
    )(_body)
    return k(x, token_table, pos_table)


def kernel(x, token_table, pos_table):
    return _embed(x.astype(jnp.int32), token_table, pos_table)
